# Initial kernel scaffold; baseline (speedup 1.0000x reference)
import jax
import jax.numpy as jnp
from jax.experimental import pallas as pl


def kernel(x, word_vectors):
    raise NotImplementedError("write your pallas kernel here")


# speedup vs baseline: 2.9724x; 2.9724x over previous
import functools

import jax
import jax.numpy as jnp
from jax import lax
from jax.experimental import pallas as pl
from jax.experimental.pallas import tpu as pltpu
from jax.experimental.pallas import tpu_sc as plsc

_NUM_CORES = 2
_NUM_SUBCORES = 16
_NW = _NUM_CORES * _NUM_SUBCORES
_CHUNK = 128


def _emb_body(table_hbm, idx_hbm, out_hbm, idx_v, rows_v, gsem, cpw):
    wid = lax.axis_index("s") * _NUM_CORES + lax.axis_index("c")
    c0 = wid * cpw
    pltpu.sync_copy(idx_hbm.at[wid], idx_v)

    @pl.loop(0, cpw)
    def _(j):
        pltpu.async_copy(table_hbm.at[idx_v.at[j]], rows_v, gsem).wait()
        pltpu.sync_copy(rows_v, out_hbm.at[c0 + j])


def kernel(x, word_vectors):
    batch, hist = x.shape
    vocab, dim = word_vectors.shape
    total = batch * hist
    assert total % (_NW * _CHUNK) == 0
    n_chunks = total // _CHUNK
    cpw = n_chunks // _NW

    idx3d = x.reshape(_NW, cpw, _CHUNK).astype(jnp.int32)

    run = pl.kernel(
        functools.partial(_emb_body, cpw=cpw),
        out_type=jax.ShapeDtypeStruct((n_chunks, _CHUNK, dim), jnp.float32),
        mesh=plsc.VectorSubcoreMesh(core_axis_name="c", subcore_axis_name="s"),
        scratch_types=[
            pltpu.VMEM((cpw, _CHUNK), jnp.int32),
            pltpu.VMEM((_CHUNK, dim), jnp.float32),
            pltpu.SemaphoreType.DMA,
        ],
    )
    out = run(word_vectors, idx3d)
    return out.reshape(batch, hist, dim)


# speedup vs baseline: 3.3402x; 1.1237x over previous
import functools

import jax
import jax.numpy as jnp
from jax import lax
from jax.experimental import pallas as pl
from jax.experimental.pallas import tpu as pltpu
from jax.experimental.pallas import tpu_sc as plsc

_NUM_CORES = 2
_NUM_SUBCORES = 16
_NW = _NUM_CORES * _NUM_SUBCORES
_CHUNK = 128


_NBUF = 2


def _emb_body(table_hbm, idx_hbm, out_hbm, idx_v, *scratch, cpw):
    rows = scratch[:_NBUF]
    gsems = scratch[_NBUF:2 * _NBUF]
    wsems = scratch[2 * _NBUF:3 * _NBUF]

    wid = lax.axis_index("s") * _NUM_CORES + lax.axis_index("c")
    c0 = wid * cpw
    pltpu.sync_copy(idx_hbm.at[wid], idx_v)

    for b in range(_NBUF):
        pltpu.async_copy(table_hbm.at[idx_v.at[b]], rows[b], gsems[b])

    @pl.loop(0, cpw // _NBUF - 1)
    def _(i):
        for b in range(_NBUF):
            j = i * _NBUF + b
            pltpu.make_async_copy(table_hbm.at[idx_v.at[j]], rows[b], gsems[b]).wait()
            pltpu.async_copy(rows[b], out_hbm.at[c0 + j], wsems[b])
            pltpu.make_async_copy(rows[b], out_hbm.at[c0 + j], wsems[b]).wait()
            pltpu.async_copy(table_hbm.at[idx_v.at[j + _NBUF]], rows[b], gsems[b])

    for b in range(_NBUF):
        j = cpw - _NBUF + b
        pltpu.make_async_copy(table_hbm.at[idx_v.at[j]], rows[b], gsems[b]).wait()
        pltpu.async_copy(rows[b], out_hbm.at[c0 + j], wsems[b])
    for b in range(_NBUF):
        j = cpw - _NBUF + b
        pltpu.make_async_copy(rows[b], out_hbm.at[c0 + j], wsems[b]).wait()


def kernel(x, word_vectors):
    batch, hist = x.shape
    vocab, dim = word_vectors.shape
    total = batch * hist
    assert total % (_NW * _CHUNK) == 0
    n_chunks = total // _CHUNK
    cpw = n_chunks // _NW

    idx3d = x.reshape(_NW, cpw, _CHUNK).astype(jnp.int32)

    run = pl.kernel(
        functools.partial(_emb_body, cpw=cpw),
        out_type=jax.ShapeDtypeStruct((n_chunks, _CHUNK, dim), jnp.float32),
        mesh=plsc.VectorSubcoreMesh(core_axis_name="c", subcore_axis_name="s"),
        scratch_types=(
            [pltpu.VMEM((cpw, _CHUNK), jnp.int32)]
            + [pltpu.VMEM((_CHUNK, dim), jnp.float32)] * _NBUF
            + [pltpu.SemaphoreType.DMA] * (2 * _NBUF)
        ),
    )
    out = run(word_vectors, idx3d)
    return out.reshape(batch, hist, dim)


# speedup vs baseline: 3.3481x; 1.0024x over previous
import functools

import jax
import jax.numpy as jnp
from jax import lax
from jax.experimental import pallas as pl
from jax.experimental.pallas import tpu as pltpu
from jax.experimental.pallas import tpu_sc as plsc

_NUM_CORES = 2
_NUM_SUBCORES = 16
_NW = _NUM_CORES * _NUM_SUBCORES
_CHUNK = 128


_NBUF = 5


def _emb_body(table_hbm, idx_hbm, out_hbm, idx_v, *scratch, cpw):
    rows = scratch[:_NBUF]
    gsems = scratch[_NBUF:2 * _NBUF]
    wsems = scratch[2 * _NBUF:3 * _NBUF]

    wid = lax.axis_index("s") * _NUM_CORES + lax.axis_index("c")
    c0 = wid * cpw
    pltpu.sync_copy(idx_hbm.at[wid], idx_v)

    for b in range(_NBUF):
        pltpu.async_copy(table_hbm.at[idx_v.at[b]], rows[b], gsems[b])

    @pl.loop(0, cpw // _NBUF - 1)
    def _(i):
        for b in range(_NBUF):
            j = i * _NBUF + b
            pltpu.make_async_copy(table_hbm.at[idx_v.at[j]], rows[b], gsems[b]).wait()
            pltpu.async_copy(rows[b], out_hbm.at[c0 + j], wsems[b])
            pltpu.make_async_copy(rows[b], out_hbm.at[c0 + j], wsems[b]).wait()
            pltpu.async_copy(table_hbm.at[idx_v.at[j + _NBUF]], rows[b], gsems[b])

    for b in range(_NBUF):
        j = cpw - _NBUF + b
        pltpu.make_async_copy(table_hbm.at[idx_v.at[j]], rows[b], gsems[b]).wait()
        pltpu.async_copy(rows[b], out_hbm.at[c0 + j], wsems[b])
    for b in range(_NBUF):
        j = cpw - _NBUF + b
        pltpu.make_async_copy(rows[b], out_hbm.at[c0 + j], wsems[b]).wait()


def kernel(x, word_vectors):
    batch, hist = x.shape
    vocab, dim = word_vectors.shape
    total = batch * hist
    assert total % (_NW * _CHUNK) == 0
    n_chunks = total // _CHUNK
    cpw = n_chunks // _NW

    idx3d = x.reshape(_NW, cpw, _CHUNK).astype(jnp.int32)

    run = pl.kernel(
        functools.partial(_emb_body, cpw=cpw),
        out_type=jax.ShapeDtypeStruct((n_chunks, _CHUNK, dim), jnp.float32),
        mesh=plsc.VectorSubcoreMesh(core_axis_name="c", subcore_axis_name="s"),
        scratch_types=(
            [pltpu.VMEM((cpw, _CHUNK), jnp.int32)]
            + [pltpu.VMEM((_CHUNK, dim), jnp.float32)] * _NBUF
            + [pltpu.SemaphoreType.DMA] * (2 * _NBUF)
        ),
    )
    out = run(word_vectors, idx3d)
    return out.reshape(batch, hist, dim)


# speedup vs baseline: 5.9165x; 1.7671x over previous
import functools

import jax
import jax.numpy as jnp
from jax import lax
from jax.experimental import pallas as pl
from jax.experimental.pallas import tpu as pltpu
from jax.experimental.pallas import tpu_sc as plsc

_NUM_CORES = 2
_NUM_SUBCORES = 16
_NW = _NUM_CORES * _NUM_SUBCORES
_G = 4
_NBUF = 2


def _emb_body(table_hbm, idx_hbm, out_hbm, idx_v, *scratch, rpw, hist):
    rows = scratch[:_NBUF]
    gsems = scratch[_NBUF:2 * _NBUF]
    wsems = scratch[2 * _NBUF:3 * _NBUF]
    ngroups = rpw // _G

    wid = lax.axis_index("s") * _NUM_CORES + lax.axis_index("c")
    r0 = wid * rpw
    pltpu.sync_copy(idx_hbm.at[pl.ds(r0, rpw)], idx_v)

    def start_gathers(g, b):
        for r in range(_G):
            pltpu.async_copy(table_hbm.at[idx_v.at[g * _G + r]], rows[b].at[r],
                             gsems[b])

    def wait_gathers(g, b):
        for r in range(_G):
            pltpu.make_async_copy(table_hbm.at[idx_v.at[g * _G + r]],
                                  rows[b].at[r], gsems[b]).wait()

    def start_wb(g, b):
        pltpu.async_copy(rows[b], out_hbm.at[pl.ds(r0 + g * _G, _G)], wsems[b])

    def wait_wb(g, b):
        pltpu.make_async_copy(rows[b], out_hbm.at[pl.ds(r0 + g * _G, _G)],
                              wsems[b]).wait()

    for b in range(_NBUF):
        start_gathers(b, b)

    @pl.loop(0, ngroups // _NBUF - 1)
    def _(i):
        for b in range(_NBUF):
            g = i * _NBUF + b
            wait_gathers(g, b)
            start_wb(g, b)
            wait_wb(g, b)
            start_gathers(g + _NBUF, b)

    for b in range(_NBUF):
        g = ngroups - _NBUF + b
        wait_gathers(g, b)
        start_wb(g, b)
    for b in range(_NBUF):
        wait_wb(ngroups - _NBUF + b, b)


def kernel(x, word_vectors):
    batch, hist = x.shape
    vocab, dim = word_vectors.shape
    assert batch % (_NW * _G) == 0
    rpw = batch // _NW

    idx = x.astype(jnp.int32)

    run = pl.kernel(
        functools.partial(_emb_body, rpw=rpw, hist=hist),
        out_type=jax.ShapeDtypeStruct((batch, hist, dim), jnp.float32),
        mesh=plsc.VectorSubcoreMesh(core_axis_name="c", subcore_axis_name="s"),
        scratch_types=(
            [pltpu.VMEM((rpw, hist), jnp.int32)]
            + [pltpu.VMEM((_G, hist, dim), jnp.float32)] * _NBUF
            + [pltpu.SemaphoreType.DMA] * (2 * _NBUF)
        ),
    )
    return run(word_vectors, idx)


# speedup vs baseline: 5.9443x; 1.0047x over previous
import functools

import jax
import jax.numpy as jnp
from jax import lax
from jax.experimental import pallas as pl
from jax.experimental.pallas import tpu as pltpu
from jax.experimental.pallas import tpu_sc as plsc

_NUM_CORES = 2
_NUM_SUBCORES = 16
_NW = _NUM_CORES * _NUM_SUBCORES
_G = 4
_NBUF = 2


def _emb_body(table_hbm, idx_hbm, out_hbm, idx_v, *scratch, rpw, hist):
    rows = scratch[:_NBUF]
    gsems = scratch[_NBUF:2 * _NBUF]
    wsems = scratch[2 * _NBUF:3 * _NBUF]
    ngroups = rpw // _G

    wid = lax.axis_index("s") * _NUM_CORES + lax.axis_index("c")
    r0 = wid * rpw
    pltpu.sync_copy(idx_hbm.at[pl.ds(r0, rpw)], idx_v)

    def start_gathers(g, b):
        for r in range(_G):
            pltpu.async_copy(table_hbm.at[idx_v.at[g * _G + r]], rows[b].at[r],
                             gsems[b])

    def wait_gathers(g, b):
        for r in range(_G):
            pltpu.make_async_copy(table_hbm.at[idx_v.at[g * _G + r]],
                                  rows[b].at[r], gsems[b]).wait()

    def start_wb(g, b):
        pltpu.async_copy(rows[b], out_hbm.at[pl.ds(r0 + g * _G, _G)], wsems[b])

    def wait_wb(g, b):
        pltpu.make_async_copy(rows[b], out_hbm.at[pl.ds(r0 + g * _G, _G)],
                              wsems[b]).wait()

    for b in range(_NBUF):
        start_gathers(b, b)

    @pl.loop(0, ngroups // _NBUF - 1)
    def _(i):
        for b in range(_NBUF):
            g = i * _NBUF + b
            wait_gathers(g, b)
            start_wb(g, b)
            wait_wb(g, b)
            start_gathers(g + _NBUF, b)

    for b in range(_NBUF):
        g = ngroups - _NBUF + b
        wait_gathers(g, b)
        start_wb(g, b)
    for b in range(_NBUF):
        wait_wb(ngroups - _NBUF + b, b)


def kernel(x, word_vectors):
    batch, hist = x.shape
    vocab, dim = word_vectors.shape
    assert batch % (_NW * _G) == 0
    rpw = batch // _NW

    idx = x.astype(jnp.int32)

    run = pl.kernel(
        functools.partial(_emb_body, rpw=rpw, hist=hist),
        out_type=jax.ShapeDtypeStruct((batch, hist, dim), jnp.float32),
        mesh=plsc.VectorSubcoreMesh(core_axis_name="c", subcore_axis_name="s"),
        compiler_params=pltpu.CompilerParams(use_tc_tiling_on_sc=True),
        scratch_types=(
            [pltpu.VMEM((rpw, hist), jnp.int32)]
            + [pltpu.VMEM((_G, hist, dim), jnp.float32)] * _NBUF
            + [pltpu.SemaphoreType.DMA] * (2 * _NBUF)
        ),
    )
    return run(word_vectors, idx)


# speedup vs baseline: 10.3495x; 1.7411x over previous
import functools

import jax
import jax.numpy as jnp
from jax import lax
from jax.experimental import pallas as pl
from jax.experimental.pallas import tpu as pltpu
from jax.experimental.pallas import tpu_sc as plsc

_NUM_CORES = 2
_NUM_SUBCORES = 16
_NW = _NUM_CORES * _NUM_SUBCORES
_CHUNK = 128
_NBUF = 2


def _emb_body(table_hbm, idx_hbm, out_hbm, idx_v, *scratch, cpw):
    rows = scratch[:_NBUF]
    gsems = scratch[_NBUF:2 * _NBUF]
    wsems = scratch[2 * _NBUF:3 * _NBUF]

    wid = lax.axis_index("s") * _NUM_CORES + lax.axis_index("c")
    c0 = wid * cpw
    pltpu.sync_copy(idx_hbm.at[wid], idx_v)

    def start_gather(g, b):
        pltpu.async_copy(table_hbm.at[idx_v.at[g]], rows[b], gsems[b])

    def wait_gather(g, b):
        pltpu.make_async_copy(table_hbm.at[idx_v.at[g]], rows[b],
                              gsems[b]).wait()

    def start_wb(g, b):
        pltpu.async_copy(rows[b], out_hbm.at[pl.ds((c0 + g) * _CHUNK, _CHUNK)],
                         wsems[b])

    def wait_wb(g, b):
        pltpu.make_async_copy(rows[b],
                              out_hbm.at[pl.ds((c0 + g) * _CHUNK, _CHUNK)],
                              wsems[b]).wait()

    for b in range(_NBUF):
        start_gather(b, b)

    @pl.loop(0, cpw // _NBUF - 1)
    def _(i):
        for b in range(_NBUF):
            g = i * _NBUF + b
            wait_gather(g, b)
            start_wb(g, b)
            wait_wb(g, b)
            start_gather(g + _NBUF, b)

    for b in range(_NBUF):
        wait_gather(cpw - _NBUF + b, b)
        start_wb(cpw - _NBUF + b, b)
    for b in range(_NBUF):
        wait_wb(cpw - _NBUF + b, b)


def kernel(x, word_vectors):
    batch, hist = x.shape
    vocab, dim = word_vectors.shape
    total = batch * hist
    assert total % (_NW * _CHUNK) == 0
    cpw = total // _CHUNK // _NW
    assert cpw % _NBUF == 0 and cpw // _NBUF >= 2

    idx3d = x.T.reshape(_NW, cpw, _CHUNK).astype(jnp.int32)

    run = pl.kernel(
        functools.partial(_emb_body, cpw=cpw),
        out_type=jax.ShapeDtypeStruct((total, dim), jnp.float32),
        mesh=plsc.VectorSubcoreMesh(core_axis_name="c", subcore_axis_name="s"),
        scratch_types=(
            [pltpu.VMEM((cpw, _CHUNK), jnp.int32)]
            + [pltpu.VMEM((_CHUNK, dim), jnp.float32)] * _NBUF
            + [pltpu.SemaphoreType.DMA] * (2 * _NBUF)
        ),
    )
    out = run(word_vectors, idx3d)
    return out.reshape(hist, batch, dim).transpose(1, 0, 2)


# speedup vs baseline: 10.4073x; 1.0056x over previous
import functools

import jax
import jax.numpy as jnp
from jax import lax
from jax.experimental import pallas as pl
from jax.experimental.pallas import tpu as pltpu
from jax.experimental.pallas import tpu_sc as plsc

_NUM_CORES = 2
_NUM_SUBCORES = 16
_NW = _NUM_CORES * _NUM_SUBCORES
_CHUNK = 128
_NBUF = 5


def _emb_body(table_hbm, idx_hbm, out_hbm, idx_v, *scratch, cpw):
    rows = scratch[:_NBUF]
    gsems = scratch[_NBUF:2 * _NBUF]
    wsems = scratch[2 * _NBUF:3 * _NBUF]

    wid = lax.axis_index("s") * _NUM_CORES + lax.axis_index("c")
    c0 = wid * cpw
    pltpu.sync_copy(idx_hbm.at[wid], idx_v)

    def start_gather(g, b):
        pltpu.async_copy(table_hbm.at[idx_v.at[g]], rows[b], gsems[b])

    def wait_gather(g, b):
        pltpu.make_async_copy(table_hbm.at[idx_v.at[g]], rows[b],
                              gsems[b]).wait()

    def start_wb(g, b):
        pltpu.async_copy(rows[b], out_hbm.at[pl.ds((c0 + g) * _CHUNK, _CHUNK)],
                         wsems[b])

    def wait_wb(g, b):
        pltpu.make_async_copy(rows[b],
                              out_hbm.at[pl.ds((c0 + g) * _CHUNK, _CHUNK)],
                              wsems[b]).wait()

    for b in range(_NBUF):
        start_gather(b, b)

    @pl.loop(0, cpw // _NBUF - 1)
    def _(i):
        for b in range(_NBUF):
            g = i * _NBUF + b
            wait_gather(g, b)
            start_wb(g, b)
            wait_wb(g, b)
            start_gather(g + _NBUF, b)

    for b in range(_NBUF):
        wait_gather(cpw - _NBUF + b, b)
        start_wb(cpw - _NBUF + b, b)
    for b in range(_NBUF):
        wait_wb(cpw - _NBUF + b, b)


def kernel(x, word_vectors):
    batch, hist = x.shape
    vocab, dim = word_vectors.shape
    total = batch * hist
    assert total % (_NW * _CHUNK) == 0
    cpw = total // _CHUNK // _NW
    assert cpw % _NBUF == 0 and cpw // _NBUF >= 2

    idx3d = x.T.reshape(_NW, cpw, _CHUNK).astype(jnp.int32)

    run = pl.kernel(
        functools.partial(_emb_body, cpw=cpw),
        out_type=jax.ShapeDtypeStruct((total, dim), jnp.float32),
        mesh=plsc.VectorSubcoreMesh(core_axis_name="c", subcore_axis_name="s"),
        scratch_types=(
            [pltpu.VMEM((cpw, _CHUNK), jnp.int32)]
            + [pltpu.VMEM((_CHUNK, dim), jnp.float32)] * _NBUF
            + [pltpu.SemaphoreType.DMA] * (2 * _NBUF)
        ),
    )
    out = run(word_vectors, idx3d)
    return out.reshape(hist, batch, dim).transpose(1, 0, 2)
